# baseline (speedup 1.0000x reference)
import functools

import jax
import jax.numpy as jnp
from jax.experimental import pallas as pl
from jax.experimental.pallas import tpu as pltpu

DEPTH = 16
NSEM = 4


@functools.lru_cache(maxsize=None)
def _make_tc_gather(B, V, D):
    assert DEPTH % NSEM == 0 and B % NSEM == 0 and (B - 2 * DEPTH) % NSEM == 0

    def body(idx_ref, table_ref, out_ref, *sems):
        def start(t, sem):
            r = idx_ref[t]
            pltpu.make_async_copy(table_ref.at[r], out_ref.at[t], sem).start()

        def wait(t, sem):
            pltpu.make_async_copy(table_ref.at[0], out_ref.at[t], sem).wait()

        @pl.loop(0, DEPTH, step=NSEM)
        def _(t):
            for j in range(NSEM):
                start(t + j, sems[j])

        @pl.loop(DEPTH, B, step=NSEM)
        def _(t):
            for j in range(NSEM):
                start(t + j, sems[j])
                wait(t + j - DEPTH, sems[j])

        @pl.loop(B - DEPTH, B, step=NSEM)
        def _(t):
            for j in range(NSEM):
                wait(t + j, sems[j])

    grid_spec = pltpu.PrefetchScalarGridSpec(
        num_scalar_prefetch=1,
        grid=(1,),
        in_specs=[pl.BlockSpec(memory_space=pl.ANY)],
        out_specs=pl.BlockSpec(memory_space=pl.ANY),
        scratch_shapes=[pltpu.SemaphoreType.DMA] * NSEM,
    )
    return pl.pallas_call(
        body,
        grid_spec=grid_spec,
        out_shape=jax.ShapeDtypeStruct((B, D), jnp.float32),
    )


def kernel(input_ids, embed_table):
    batch, seq = input_ids.shape
    vocab, d = embed_table.shape
    idx = input_ids.reshape(-1).astype(jnp.int32)
    out = _make_tc_gather(batch * seq, vocab, d)(idx, embed_table)
    return out.reshape(batch, seq, d)
